# baseline (speedup 1.0000x reference)
import functools

import jax
import jax.numpy as jnp
from jax import lax
from jax.experimental import pallas as pl
from jax.experimental.pallas import tpu as pltpu
from jax.experimental.pallas import tpu_sc as plsc

N = 10000
E = 160000
D = 256
DH = 128
N_PAD = 10240
TRASH = N
NSUB = 16
CHUNK = 128
CH_PER_SUB = 80
EDGES_PER_SUB = CH_PER_SUB * CHUNK
E_PAD = NSUB * EDGES_PER_SUB
ROWS_PER_SUB = N_PAD // NSUB
NGRP = CH_PER_SUB // 4

_MESH = plsc.VectorSubcoreMesh(core_axis_name="c", subcore_axis_name="s")


@functools.partial(
    pl.kernel,
    out_type=(jax.ShapeDtypeStruct((2, N_PAD, DH), jnp.float32),
              jax.ShapeDtypeStruct((NSUB, N_PAD), jnp.float32)),
    mesh=_MESH,
    compiler_params=pltpu.CompilerParams(needs_layout_passes=False),
    scratch_types=[
        pltpu.VMEM((4, CHUNK), jnp.int32),
        pltpu.VMEM((4, CHUNK), jnp.int32),
        pltpu.VMEM((CHUNK, DH), jnp.float32),
        pltpu.VMEM((CHUNK, DH), jnp.float32),
        pltpu.VMEM((N_PAD,), jnp.float32),
        pltpu.VMEM_SHARED((N_PAD, DH), jnp.float32),
        pltpu.SemaphoreType.DMA,
        pltpu.SemaphoreType.DMA,
        pltpu.SemaphoreType.DMA,
        pltpu.SemaphoreType.DMA,
        pltpu.SemaphoreType.DMA,
        pltpu.SemaphoreType.DMA,
        pltpu.SemaphoreType.DMA,
        pltpu.SemaphoreType.DMA,
    ],
)
def _segsum(tables_hbm, src2_hbm, dst_hbm, out_hbm, c16_hbm,
            sidx_v, didx_v, rows0, rows1, cnt_v, acc_sh,
            ga, gb, sa, sb, i0, i1, i2, i3):
    ci = lax.axis_index("c")
    si = lax.axis_index("s")
    zv = jnp.zeros((16,), jnp.float32)
    ones16 = jnp.ones((16,), jnp.float32)
    rows = [rows0, rows1]
    gsem = [ga, gb]
    ssem = [sa, sb]
    isem = [i0, i1, i2, i3]

    def idx_copy(c, slot):
        pltpu.async_copy(src2_hbm.at[ci, si, c], sidx_v.at[slot], isem[slot])
        pltpu.async_copy(dst_hbm.at[si, c], didx_v.at[slot], isem[slot])

    def idx_wait(c, slot):
        pltpu.make_async_copy(src2_hbm.at[ci, si, c], sidx_v.at[slot],
                              isem[slot]).wait()
        pltpu.make_async_copy(dst_hbm.at[si, c], didx_v.at[slot],
                              isem[slot]).wait()

    def gather(slot, b):
        pltpu.async_copy(tables_hbm.at[sidx_v.at[slot]], rows[b], gsem[b])

    def gather_wait(slot, b):
        pltpu.make_async_copy(tables_hbm.at[sidx_v.at[slot]], rows[b],
                              gsem[b]).wait()

    for slot in range(4):
        idx_copy(slot, slot)

    with jax.named_scope("sc_zero"):
        @pl.loop(0, CHUNK)
        def _(r):
            @pl.loop(0, DH // 16)
            def _(cc):
                rows0[r, pl.ds(cc * 16, 16)] = zv

        @pl.loop(0, N_PAD // 16)
        def _(k):
            cnt_v[pl.ds(k * 16, 16)] = zv

        @pl.loop(0, ROWS_PER_SUB // CHUNK)
        def _(k):
            pltpu.sync_copy(rows0, acc_sh.at[pl.ds(si * ROWS_PER_SUB + k * CHUNK, CHUNK)])

    for b in range(2):
        idx_wait(b, b)
        gather(b, b)
    with jax.named_scope("sc_barrier0"):
        plsc.subcore_barrier()

    @pl.loop(0, NGRP)
    @jax.named_scope("sc_mainloop")
    def _(g):
        c0 = g * 4
        for half in range(2):
            scats = []
            for b in range(2):
                slot = 2 * half + b
                c = c0 + slot
                gather_wait(slot, b)
                scats.append(pltpu.async_copy(rows[b], acc_sh.at[didx_v.at[slot]],
                                              ssem[b], add=True))

                @pl.loop(0, CHUNK // 16)
                def _(j):
                    idx16 = didx_v[slot, pl.ds(j * 16, 16)]
                    plsc.addupdate_scatter(cnt_v, [idx16], ones16)

            for b in range(2):
                slot = 2 * half + b
                nslot = (slot + 2) % 4
                scats[b].wait()
                idx_copy((c0 + slot + 4) % CH_PER_SUB, slot)
                idx_wait((c0 + slot + 2) % CH_PER_SUB, nslot)
                gather(nslot, b)

    with jax.named_scope("sc_tail"):
        for b in range(2):
            gather_wait(b, b)
            idx_wait((b + 2) % 4, (b + 2) % 4)

        @pl.when(ci == 0)
        def _():
            pltpu.sync_copy(cnt_v, c16_hbm.at[si])

        plsc.subcore_barrier()
        pltpu.sync_copy(acc_sh.at[pl.ds(si * ROWS_PER_SUB, ROWS_PER_SUB)],
                        out_hbm.at[ci, pl.ds(si * ROWS_PER_SUB, ROWS_PER_SUB)])


BLK = 1024


def _tc_body(sums_ref, c16_ref, x_ref, wl_ref, wr_ref, b_ref, out_ref):
    s0 = sums_ref[0]
    s1 = sums_ref[1]
    cnt = jnp.sum(c16_ref[...], axis=0, keepdims=True)
    inv = (1.0 / jnp.maximum(cnt, 1.0)).reshape(BLK, 1)
    agg = jnp.concatenate([s0, s1], axis=1) * inv
    acc = jnp.dot(agg, wl_ref[...], preferred_element_type=jnp.float32)
    acc = acc + jnp.dot(x_ref[...], wr_ref[...], preferred_element_type=jnp.float32)
    out_ref[...] = acc + b_ref[...]


def kernel(x, edge_index, W_l, W_r, b):
    x = x.astype(jnp.float32)
    src = edge_index[0].astype(jnp.int32)
    dst = edge_index[1].astype(jnp.int32)

    rpad = jnp.zeros((N_PAD - N, DH), jnp.float32)
    tables = jnp.concatenate([x[:, :DH], rpad, x[:, DH:], rpad], axis=0)

    epad = jnp.full((E_PAD - E,), TRASH, jnp.int32)
    src_p = jnp.concatenate([src, epad])
    dst_p = jnp.concatenate([dst, epad]).reshape(NSUB, CH_PER_SUB, CHUNK)
    src2 = jnp.stack([src_p, src_p + N_PAD])
    src2 = src2.reshape(2, NSUB, CH_PER_SUB, CHUNK)

    sums, counts16 = _segsum(tables, src2, dst_p)

    return pl.pallas_call(
        _tc_body,
        grid=(pl.cdiv(N, BLK),),
        in_specs=[
            pl.BlockSpec((2, BLK, DH), lambda i: (0, i, 0)),
            pl.BlockSpec((NSUB, BLK), lambda i: (0, i)),
            pl.BlockSpec((BLK, D), lambda i: (i, 0)),
            pl.BlockSpec((D, D), lambda i: (0, 0)),
            pl.BlockSpec((D, D), lambda i: (0, 0)),
            pl.BlockSpec((1, D), lambda i: (0, 0)),
        ],
        out_specs=pl.BlockSpec((BLK, D), lambda i: (i, 0)),
        out_shape=jax.ShapeDtypeStruct((N, D), jnp.float32),
    )(sums, counts16, x, W_l, W_r, b.reshape(1, D))


# speedup vs baseline: 1.1023x; 1.1023x over previous
import functools

import jax
import jax.numpy as jnp
from jax import lax
from jax.experimental import pallas as pl
from jax.experimental.pallas import tpu as pltpu
from jax.experimental.pallas import tpu_sc as plsc

N = 10000
E = 160000
D = 256
DH = 128
N_PAD = 10240
TRASH = N
NSUB = 16
CHUNK = 128
CH_PER_SUB = 80
EDGES_PER_SUB = CH_PER_SUB * CHUNK
E_PAD = NSUB * EDGES_PER_SUB
ROWS_PER_SUB = N_PAD // NSUB
NGRP = CH_PER_SUB // 4

_MESH = plsc.VectorSubcoreMesh(core_axis_name="c", subcore_axis_name="s")


@functools.partial(
    pl.kernel,
    out_type=(jax.ShapeDtypeStruct((2, N_PAD, DH), jnp.float32),
              jax.ShapeDtypeStruct((NSUB, N_PAD), jnp.float32)),
    mesh=_MESH,
    compiler_params=pltpu.CompilerParams(needs_layout_passes=False),
    scratch_types=[
        pltpu.VMEM((4, CHUNK), jnp.int32),
        pltpu.VMEM((4, CHUNK), jnp.int32),
        pltpu.VMEM((CHUNK, DH), jnp.float32),
        pltpu.VMEM((CHUNK, DH), jnp.float32),
        pltpu.VMEM((N_PAD,), jnp.float32),
        pltpu.VMEM_SHARED((N_PAD, DH), jnp.float32),
        pltpu.SemaphoreType.DMA,
        pltpu.SemaphoreType.DMA,
        pltpu.SemaphoreType.DMA,
        pltpu.SemaphoreType.DMA,
        pltpu.SemaphoreType.DMA,
        pltpu.SemaphoreType.DMA,
        pltpu.SemaphoreType.DMA,
        pltpu.SemaphoreType.DMA,
    ],
)
def _segsum(tables_hbm, src2_hbm, dst_hbm, zr_hbm, zc_hbm, out_hbm, c16_hbm,
            sidx_v, didx_v, rows0, rows1, cnt_v, acc_sh,
            ga, gb, sa, sb, i0, i1, i2, i3):
    ci = lax.axis_index("c")
    si = lax.axis_index("s")
    ones16 = jnp.ones((16,), jnp.float32)
    rows = [rows0, rows1]
    gsem = [ga, gb]
    ssem = [sa, sb]
    isem = [i0, i1, i2, i3]

    def idx_copy(c, slot):
        pltpu.async_copy(src2_hbm.at[ci, si, c], sidx_v.at[slot], isem[slot])
        pltpu.async_copy(dst_hbm.at[si, c], didx_v.at[slot], isem[slot])

    def idx_wait(c, slot):
        pltpu.make_async_copy(src2_hbm.at[ci, si, c], sidx_v.at[slot],
                              isem[slot]).wait()
        pltpu.make_async_copy(dst_hbm.at[si, c], didx_v.at[slot],
                              isem[slot]).wait()

    def gather(slot, b):
        pltpu.async_copy(tables_hbm.at[sidx_v.at[slot]], rows[b], gsem[b])

    def gather_wait(slot, b):
        pltpu.make_async_copy(tables_hbm.at[sidx_v.at[slot]], rows[b],
                              gsem[b]).wait()

    for slot in range(4):
        idx_copy(slot, slot)
    pz = pltpu.async_copy(zr_hbm, rows0, ga)
    for k in range(N_PAD // 2048):
        pltpu.async_copy(zc_hbm, cnt_v.at[pl.ds(k * 2048, 2048)], gb)
    pz.wait()
    for k in range(N_PAD // 2048):
        pltpu.make_async_copy(zc_hbm, cnt_v.at[pl.ds(k * 2048, 2048)], gb).wait()

    @pl.loop(0, ROWS_PER_SUB // CHUNK)
    def _(k):
        pltpu.sync_copy(rows0, acc_sh.at[pl.ds(si * ROWS_PER_SUB + k * CHUNK, CHUNK)])

    for b in range(2):
        idx_wait(b, b)
        gather(b, b)
    plsc.subcore_barrier()

    @pl.loop(0, NGRP)
    def _(g):
        c0 = g * 4
        for half in range(2):
            scats = []
            for b in range(2):
                slot = 2 * half + b
                c = c0 + slot
                gather_wait(slot, b)
                scats.append(pltpu.async_copy(rows[b], acc_sh.at[didx_v.at[slot]],
                                              ssem[b], add=True))

                @pl.loop(0, CHUNK // 16)
                def _(j):
                    idx16 = didx_v[slot, pl.ds(j * 16, 16)]
                    plsc.addupdate_scatter(cnt_v, [idx16], ones16)

            for b in range(2):
                slot = 2 * half + b
                nslot = (slot + 2) % 4
                scats[b].wait()
                idx_copy((c0 + slot + 4) % CH_PER_SUB, slot)
                idx_wait((c0 + slot + 2) % CH_PER_SUB, nslot)
                gather(nslot, b)

    for b in range(2):
        gather_wait(b, b)
        idx_wait((b + 2) % 4, (b + 2) % 4)

    @pl.when(ci == 0)
    def _():
        pltpu.sync_copy(cnt_v, c16_hbm.at[si])

    plsc.subcore_barrier()
    pltpu.sync_copy(acc_sh.at[pl.ds(si * ROWS_PER_SUB, ROWS_PER_SUB)],
                    out_hbm.at[ci, pl.ds(si * ROWS_PER_SUB, ROWS_PER_SUB)])


BLK = 1024


def _tc_body(sums_ref, c16_ref, x_ref, wl_ref, wr_ref, b_ref, out_ref):
    s0 = sums_ref[0]
    s1 = sums_ref[1]
    cnt = jnp.sum(c16_ref[...], axis=0, keepdims=True)
    inv = (1.0 / jnp.maximum(cnt, 1.0)).reshape(BLK, 1)
    agg = jnp.concatenate([s0, s1], axis=1) * inv
    acc = jnp.dot(agg, wl_ref[...], preferred_element_type=jnp.float32)
    acc = acc + jnp.dot(x_ref[...], wr_ref[...], preferred_element_type=jnp.float32)
    out_ref[...] = acc + b_ref[...]


def kernel(x, edge_index, W_l, W_r, b):
    x = x.astype(jnp.float32)
    src = edge_index[0].astype(jnp.int32)
    dst = edge_index[1].astype(jnp.int32)

    tables = x.reshape(2 * N, DH)
    src_p = jnp.concatenate([src, jnp.zeros((E_PAD - E,), jnp.int32)])
    dst_p = jnp.concatenate([dst, jnp.full((E_PAD - E,), TRASH, jnp.int32)])
    dst_p = dst_p.reshape(NSUB, CH_PER_SUB, CHUNK)
    src2 = jnp.stack([2 * src_p, 2 * src_p + 1])
    src2 = src2.reshape(2, NSUB, CH_PER_SUB, CHUNK)
    zr = jnp.zeros((CHUNK, DH), jnp.float32)
    zc = jnp.zeros((2048,), jnp.float32)

    sums, counts16 = _segsum(tables, src2, dst_p, zr, zc)

    return pl.pallas_call(
        _tc_body,
        grid=(pl.cdiv(N, BLK),),
        in_specs=[
            pl.BlockSpec((2, BLK, DH), lambda i: (0, i, 0)),
            pl.BlockSpec((NSUB, BLK), lambda i: (0, i)),
            pl.BlockSpec((BLK, D), lambda i: (i, 0)),
            pl.BlockSpec((D, D), lambda i: (0, 0)),
            pl.BlockSpec((D, D), lambda i: (0, 0)),
            pl.BlockSpec((1, D), lambda i: (0, 0)),
        ],
        out_specs=pl.BlockSpec((BLK, D), lambda i: (i, 0)),
        out_shape=jax.ShapeDtypeStruct((N, D), jnp.float32),
    )(sums, counts16, x, W_l, W_r, b.reshape(1, D))
